# baseline (speedup 1.0000x reference)
import functools

import jax
import jax.numpy as jnp
from jax import lax
from jax.experimental import pallas as pl
from jax.experimental.pallas import tpu as pltpu
from jax.experimental.pallas import tpu_sc as plsc

N = 100000
E = 6400000
G = 256
L = 16
NCORES = 2
NSUB = 16
NW = NCORES * NSUB

RB = 16
CHUNK = 128
ROWS_PER_W = 1568
NBLK = ROWS_PER_W // RB
E2 = ROWS_PER_W * CHUNK * NW
ZR = 400
NZCH = N // ZR


def _leaky(x):
    return jnp.where(x > 0, x, 0.01 * x)


def _make_sc_agg(S):
    mesh = plsc.VectorSubcoreMesh(core_axis_name="c", subcore_axis_name="s")

    def body(*refs):
        tables = refs[:S]
        src2, dst2, ew2, out = refs[S:S + 4]
        (src_v, dst_v, ew_v, gbuf, sbuf, zero_v, acc,
         g0, g1, g2, g3, s0, s1, s2, s3, i0, i1, i2) = refs[S + 4:]
        gsem = [g0, g1, g2, g3]
        ssem = [s0, s1, s2, s3]
        c = lax.axis_index("c")
        s = lax.axis_index("s")
        w = s * NCORES + c

        def zinit(i, carry):
            zero_v[i, :] = jnp.zeros((L,), jnp.float32)
            return carry
        lax.fori_loop(0, ZR, zinit, 0)

        for j in range(S):
            def zchunk(t, carry):
                idx = s + t * NSUB

                @pl.when(idx < NZCH)
                def _():
                    pltpu.sync_copy(zero_v, acc.at[pl.ds(idx * ZR, ZR)])
                return carry
            lax.fori_loop(0, (NZCH + NSUB - 1) // NSUB, zchunk, 0)
            plsc.subcore_barrier()

            def blk_body(blk, carry):
                row0 = w * ROWS_PER_W + blk * RB
                c0 = pltpu.async_copy(src2.at[pl.ds(row0, RB)], src_v, i0)
                c1 = pltpu.async_copy(dst2.at[pl.ds(row0, RB)], dst_v, i1)
                c2 = pltpu.async_copy(ew2.at[pl.ds(row0, RB)], ew_v, i2)
                c0.wait()
                c1.wait()
                c2.wait()
                gd = [None] * 4
                sd = [None] * 4
                for r in range(4):
                    gd[r] = pltpu.async_copy(
                        tables[j].at[src_v.at[r]], gbuf.at[r], gsem[r])
                for r in range(RB):
                    b = r % 4
                    gd[b].wait()
                    if sd[b] is not None:
                        sd[b].wait()

                    def scale(g):
                        ewg = ew_v[r, pl.ds(g * 16, 16)]
                        for lane in range(16):
                            e = g * 16 + lane
                            sbuf[b, e, :] = gbuf[b, e, :] * ewg[lane]
                    plsc.parallel_loop(0, CHUNK // 16, 1, unroll=4)(scale)

                    if r + 4 < RB:
                        gd[b] = pltpu.async_copy(
                            tables[j].at[src_v.at[r + 4]], gbuf.at[b], gsem[b])
                    sd[b] = pltpu.async_copy(
                        sbuf.at[b], acc.at[dst_v.at[r]], ssem[b], add=True)
                for b in range(4):
                    sd[b].wait()
                return carry
            lax.fori_loop(0, NBLK, blk_body, 0)
            plsc.subcore_barrier()

            def fchunk(t, carry):
                idx = s + t * NSUB

                @pl.when(idx < NZCH)
                def _():
                    start = (2 * j + c) * N + idx * ZR
                    pltpu.sync_copy(acc.at[pl.ds(idx * ZR, ZR)],
                                    out.at[pl.ds(start, ZR)])
                return carry
            lax.fori_loop(0, (NZCH + NSUB - 1) // NSUB, fchunk, 0)

    table_types = [jax.ShapeDtypeStruct((N, L), jnp.float32)] * S
    del table_types
    kern = pl.kernel(
        body,
        out_type=jax.ShapeDtypeStruct((2 * S * N, L), jnp.float32),
        mesh=mesh,
        scratch_types=[
            pltpu.VMEM((RB, CHUNK), jnp.int32),
            pltpu.VMEM((RB, CHUNK), jnp.int32),
            pltpu.VMEM((RB, CHUNK), jnp.float32),
            pltpu.VMEM((4, CHUNK, L), jnp.float32),
            pltpu.VMEM((4, CHUNK, L), jnp.float32),
            pltpu.VMEM((ZR, L), jnp.float32),
            pltpu.VMEM_SHARED((N, L), jnp.float32),
        ] + [pltpu.SemaphoreType.DMA] * 11,
        compiler_params=pltpu.CompilerParams(use_tc_tiling_on_sc=False),
    )
    return kern


BN_TC = 2000
NG_TC = N // BN_TC


def _tc1_body(p_ref, W_ref, b_ref, o_ref):
    agg = p_ref[0] + p_ref[1]
    h = jnp.dot(agg, W_ref[...].T, preferred_element_type=jnp.float32)
    o_ref[...] = _leaky(h + b_ref[...])


def _tc2_body(p_ref, W_ref, b_ref, oa_ref, ob_ref):
    agg = p_ref[0] + p_ref[1]
    h = jnp.dot(agg, W_ref[...].T, preferred_element_type=jnp.float32)
    h = _leaky(h + b_ref[...])
    oa_ref[...] = h[:, :L]
    ob_ref[...] = h[:, L:]


def _tc3_body(p_ref, W3_ref, b3_ref, W4_ref, o0, o1, o2, o3):
    h2 = jnp.concatenate([p_ref[0] + p_ref[1], p_ref[2] + p_ref[3]], axis=1)
    h3 = jnp.dot(h2, W3_ref[...].T, preferred_element_type=jnp.float32)
    h3 = _leaky(h3 + b3_ref[...])
    v4 = jnp.dot(h3, W4_ref[...].T, preferred_element_type=jnp.float32)
    for q, o in enumerate((o0, o1, o2, o3)):
        o[...] = v4[:, q * L:(q + 1) * L]


def _tc4_body(p_ref, b4_ref, oh_ref, os_ref):
    i = pl.program_id(0)
    h4 = jnp.concatenate(
        [p_ref[2 * k] + p_ref[2 * k + 1] for k in range(4)], axis=1)
    h4 = h4 + b4_ref[...]
    oh_ref[...] = h4

    @pl.when(i == 0)
    def _():
        os_ref[...] = jnp.zeros((8, 4 * L), jnp.float32)
    os_ref[0:1, :] += jnp.sum(h4, axis=0, keepdims=True)


def _tc4c_body(h_ref, sum_ref, gb_ref, oss_ref, scr):
    i = pl.program_id(0)
    mu = sum_ref[0:1, :] * (1.0 / N)
    d = h_ref[...] - mu
    ssq = jnp.sum(d * d, axis=0, keepdims=True)

    @pl.when(i == 0)
    def _():
        scr[...] = jnp.zeros((8, 4 * L), jnp.float32)
    scr[0:1, :] += ssq

    @pl.when(i == NG_TC - 1)
    def _():
        var = scr[0:1, :] * (1.0 / N)
        invstd = lax.rsqrt(var + 1e-5)
        scale = gb_ref[0:1, :] * invstd
        shift = gb_ref[1:2, :] - mu * scale
        oss_ref[...] = jnp.concatenate([scale, shift], axis=0)


def _tcp_body(h_ref, ss_ref, b_ref, o_ref):
    i = pl.program_id(0)
    hn = h_ref[...] * ss_ref[0:1, :] + ss_ref[1:2, :]
    hn = jnp.maximum(hn, hn * 0.01)
    bio = lax.broadcasted_iota(jnp.int32, (G, BN_TC), 0)
    ids = b_ref[pl.ds(i, 1), :]
    onehot = (bio == ids).astype(jnp.float32)
    pooled = jnp.dot(onehot, hn, preferred_element_type=jnp.float32)

    @pl.when(i == 0)
    def _():
        o_ref[...] = jnp.zeros((G, 4 * L), jnp.float32)
    o_ref[...] += pooled


def _tc5_body(p_ref, fW1_ref, fb1_ref, fW2_ref, fb2_ref, fW3_ref, fb3_ref,
              o_ref):
    pooled = p_ref[...]
    o = jnp.dot(pooled, fW1_ref[...].T, preferred_element_type=jnp.float32)
    o = _leaky(o + fb1_ref[...])
    o = jnp.dot(o, fW2_ref[...].T, preferred_element_type=jnp.float32)
    o = _leaky(o + fb2_ref[...])
    o = jnp.dot(o, fW3_ref[...].T, preferred_element_type=jnp.float32)
    o_ref[...] = o + fb3_ref[...]


def kernel(x, edge_index, edge_weight, batch, W1, b1, W2, b2, W3, b3, W4, b4,
           gamma, beta, fW1, fb1, fW2, fb2, fW3, fb3):
    f32 = jnp.float32
    src = edge_index[0]
    dst = edge_index[1]
    pad = E2 - E
    fill = (jnp.arange(pad, dtype=jnp.int32) * 37) % N
    src2 = jnp.concatenate([src, fill]).reshape(E2 // CHUNK, CHUNK)
    dst2 = jnp.concatenate([dst, fill]).reshape(E2 // CHUNK, CHUNK)
    ew2 = jnp.concatenate(
        [edge_weight, jnp.zeros((pad,), f32)]).reshape(E2 // CHUNK, CHUNK)

    xp = jnp.pad(x, ((0, 0), (0, L - 6)))
    W1p = jnp.pad(W1, ((0, 0), (0, L - 6)))
    W4p = jnp.pad(W4, ((0, 64 - 50), (0, 0)))
    b4p = jnp.pad(b4, (0, 64 - 50)).reshape(1, 64)
    gb = jnp.stack([jnp.pad(gamma, (0, 64 - 50)),
                    jnp.pad(beta, (0, 64 - 50))])
    fW1p = jnp.pad(fW1, ((0, 0), (0, 64 - 50)))

    agg_k1 = _make_sc_agg(1)
    agg_k2 = _make_sc_agg(2)
    agg_k4 = _make_sc_agg(4)

    grid = NG_TC

    def blk(k):
        return pl.BlockSpec((k, BN_TC, L), lambda i: (0, i, 0))

    def blk2(w):
        return pl.BlockSpec((BN_TC, w), lambda i: (i, 0))

    def full(shape):
        return pl.BlockSpec(shape, lambda i: tuple(0 for _ in shape))

    p1 = agg_k1(xp, src2, dst2, ew2).reshape(2, N, L)
    h1 = pl.pallas_call(
        _tc1_body, grid=(grid,),
        in_specs=[blk(2), full((L, L)), full((1, L))],
        out_specs=blk2(L),
        out_shape=jax.ShapeDtypeStruct((N, L), f32),
    )(p1, W1p, b1.reshape(1, L))

    p2 = agg_k1(h1, src2, dst2, ew2).reshape(2, N, L)
    h2a, h2b = pl.pallas_call(
        _tc2_body, grid=(grid,),
        in_specs=[blk(2), full((32, L)), full((1, 32))],
        out_specs=[blk2(L), blk2(L)],
        out_shape=[jax.ShapeDtypeStruct((N, L), f32)] * 2,
    )(p2, W2, b2.reshape(1, 32))

    p3 = agg_k2(h2a, h2b, src2, dst2, ew2).reshape(4, N, L)
    v4 = pl.pallas_call(
        _tc3_body, grid=(grid,),
        in_specs=[blk(4), full((64, 32)), full((1, 64)), full((64, 64))],
        out_specs=[blk2(L)] * 4,
        out_shape=[jax.ShapeDtypeStruct((N, L), f32)] * 4,
    )(p3, W3, b3.reshape(1, 64), W4p)

    p4 = agg_k4(v4[0], v4[1], v4[2], v4[3], src2, dst2, ew2).reshape(8, N, L)
    h4, sums = pl.pallas_call(
        _tc4_body, grid=(grid,),
        in_specs=[blk(8), full((1, 64))],
        out_specs=[blk2(64), full((8, 64))],
        out_shape=[jax.ShapeDtypeStruct((N, 64), f32),
                   jax.ShapeDtypeStruct((8, 64), f32)],
    )(p4, b4p)

    ss = pl.pallas_call(
        _tc4c_body, grid=(grid,),
        in_specs=[blk2(64), full((8, 64)), full((2, 64))],
        out_specs=full((2, 64)),
        out_shape=jax.ShapeDtypeStruct((2, 64), f32),
        scratch_shapes=[pltpu.VMEM((8, 64), f32)],
    )(h4, sums, gb)

    pooled = pl.pallas_call(
        _tcp_body, grid=(grid,),
        in_specs=[blk2(64), full((2, 64)), full((NG_TC, BN_TC))],
        out_specs=full((G, 64)),
        out_shape=jax.ShapeDtypeStruct((G, 64), f32),
    )(h4, ss, batch.reshape(NG_TC, BN_TC))

    out = pl.pallas_call(
        _tc5_body, grid=(1,),
        in_specs=[full((G, 64)), full((30, 64)), full((1, 30)),
                  full((20, 30)), full((1, 20)), full((2, 20)), full((1, 2))],
        out_specs=full((G, 2)),
        out_shape=jax.ShapeDtypeStruct((G, 2), f32),
    )(pooled, fW1p, fb1.reshape(1, 30), fW2, fb2.reshape(1, 20),
      fW3, fb3.reshape(1, 2))
    return out


# speedup vs baseline: 1.0511x; 1.0511x over previous
import functools

import jax
import jax.numpy as jnp
from jax import lax
from jax.experimental import pallas as pl
from jax.experimental.pallas import tpu as pltpu
from jax.experimental.pallas import tpu_sc as plsc

N = 100000
E = 6400000
G = 256
L = 16
NCORES = 2
NSUB = 16
NW = NCORES * NSUB

RB = 16
CHUNK = 128
ROWS_PER_W = 1568
NBLK = ROWS_PER_W // RB
E2 = ROWS_PER_W * CHUNK * NW
ZR = 100
NZCH = N // ZR


def _leaky(x):
    return jnp.where(x > 0, x, 0.01 * x)


def _make_sc_agg(S):
    mesh = plsc.VectorSubcoreMesh(core_axis_name="c", subcore_axis_name="s")

    def body(*refs):
        tables = refs[:S]
        src2, dst2, ew2, out = refs[S:S + 4]
        (src_v, dst_v, ew_v, gbuf, sbuf, zero_v, acc,
         g0, g1, g2, g3, s0, s1, s2, s3, i0, i1, i2) = refs[S + 4:]
        gsem = [g0, g1, g2, g3]
        ssem = [s0, s1, s2, s3]
        c = lax.axis_index("c")
        s = lax.axis_index("s")
        w = s * NCORES + c

        def zinit(i, carry):
            zero_v[i, :] = jnp.zeros((L,), jnp.float32)
            return carry
        lax.fori_loop(0, ZR, zinit, 0)

        for j in range(S):
            def zchunk(t, carry):
                idx = s + t * NSUB

                @pl.when(idx < NZCH)
                def _():
                    pltpu.sync_copy(zero_v, acc.at[pl.ds(idx * ZR, ZR)])
                return carry
            lax.fori_loop(0, (NZCH + NSUB - 1) // NSUB, zchunk, 0)
            plsc.subcore_barrier()

            wrow = w * ROWS_PER_W
            pltpu.async_copy(src2.at[pl.ds(wrow, RB)], src_v.at[0], i0)
            pltpu.async_copy(dst2.at[pl.ds(wrow, RB)], dst_v.at[0], i1)
            pltpu.async_copy(ew2.at[pl.ds(wrow, RB)], ew_v.at[0], i2)

            def blk_body(blk, carry):
                row0 = w * ROWS_PER_W + blk * RB
                bb = lax.rem(blk, 2)
                nb = 1 - bb
                pltpu.make_async_copy(
                    src2.at[pl.ds(row0, RB)], src_v.at[bb], i0).wait()
                pltpu.make_async_copy(
                    dst2.at[pl.ds(row0, RB)], dst_v.at[bb], i1).wait()
                pltpu.make_async_copy(
                    ew2.at[pl.ds(row0, RB)], ew_v.at[bb], i2).wait()

                @pl.when(blk < NBLK - 1)
                def _():
                    row1 = row0 + RB
                    pltpu.async_copy(src2.at[pl.ds(row1, RB)], src_v.at[nb], i0)
                    pltpu.async_copy(dst2.at[pl.ds(row1, RB)], dst_v.at[nb], i1)
                    pltpu.async_copy(ew2.at[pl.ds(row1, RB)], ew_v.at[nb], i2)

                gd = [None] * 4
                sd = [None] * 4
                for r in range(4):
                    gd[r] = pltpu.async_copy(
                        tables[j].at[src_v.at[bb, r]], gbuf.at[r], gsem[r])
                for r in range(RB):
                    b = r % 4
                    gd[b].wait()
                    if sd[b] is not None:
                        sd[b].wait()

                    def scale(g):
                        ewg = ew_v[bb, r, pl.ds(g * 16, 16)]
                        for lane in range(16):
                            e = g * 16 + lane
                            sbuf[b, e, :] = gbuf[b, e, :] * ewg[lane]
                    plsc.parallel_loop(0, CHUNK // 16, 1, unroll=4)(scale)

                    if r + 4 < RB:
                        gd[b] = pltpu.async_copy(
                            tables[j].at[src_v.at[bb, r + 4]], gbuf.at[b],
                            gsem[b])
                    sd[b] = pltpu.async_copy(
                        sbuf.at[b], acc.at[dst_v.at[bb, r]], ssem[b], add=True)
                for b in range(4):
                    sd[b].wait()
                return carry
            lax.fori_loop(0, NBLK, blk_body, 0)
            plsc.subcore_barrier()

            def fchunk(t, carry):
                idx = s + t * NSUB

                @pl.when(idx < NZCH)
                def _():
                    start = (2 * j + c) * N + idx * ZR
                    pltpu.sync_copy(acc.at[pl.ds(idx * ZR, ZR)],
                                    out.at[pl.ds(start, ZR)])
                return carry
            lax.fori_loop(0, (NZCH + NSUB - 1) // NSUB, fchunk, 0)

    table_types = [jax.ShapeDtypeStruct((N, L), jnp.float32)] * S
    del table_types
    kern = pl.kernel(
        body,
        out_type=jax.ShapeDtypeStruct((2 * S * N, L), jnp.float32),
        mesh=mesh,
        scratch_types=[
            pltpu.VMEM((2, RB, CHUNK), jnp.int32),
            pltpu.VMEM((2, RB, CHUNK), jnp.int32),
            pltpu.VMEM((2, RB, CHUNK), jnp.float32),
            pltpu.VMEM((4, CHUNK, L), jnp.float32),
            pltpu.VMEM((4, CHUNK, L), jnp.float32),
            pltpu.VMEM((ZR, L), jnp.float32),
            pltpu.VMEM_SHARED((N, L), jnp.float32),
        ] + [pltpu.SemaphoreType.DMA] * 11,
        compiler_params=pltpu.CompilerParams(use_tc_tiling_on_sc=False),
    )
    return kern


BN_TC = 2000
NG_TC = N // BN_TC


def _tc1_body(p_ref, W_ref, b_ref, o_ref):
    agg = p_ref[0] + p_ref[1]
    h = jnp.dot(agg, W_ref[...].T, preferred_element_type=jnp.float32)
    o_ref[...] = _leaky(h + b_ref[...])


def _tc2_body(p_ref, W_ref, b_ref, oa_ref, ob_ref):
    agg = p_ref[0] + p_ref[1]
    h = jnp.dot(agg, W_ref[...].T, preferred_element_type=jnp.float32)
    h = _leaky(h + b_ref[...])
    oa_ref[...] = h[:, :L]
    ob_ref[...] = h[:, L:]


def _tc3_body(p_ref, W3_ref, b3_ref, W4_ref, o0, o1, o2, o3):
    h2 = jnp.concatenate([p_ref[0] + p_ref[1], p_ref[2] + p_ref[3]], axis=1)
    h3 = jnp.dot(h2, W3_ref[...].T, preferred_element_type=jnp.float32)
    h3 = _leaky(h3 + b3_ref[...])
    v4 = jnp.dot(h3, W4_ref[...].T, preferred_element_type=jnp.float32)
    for q, o in enumerate((o0, o1, o2, o3)):
        o[...] = v4[:, q * L:(q + 1) * L]


def _tc4_body(p_ref, b4_ref, oh_ref, os_ref):
    i = pl.program_id(0)
    h4 = jnp.concatenate(
        [p_ref[2 * k] + p_ref[2 * k + 1] for k in range(4)], axis=1)
    h4 = h4 + b4_ref[...]
    oh_ref[...] = h4

    @pl.when(i == 0)
    def _():
        os_ref[...] = jnp.zeros((8, 4 * L), jnp.float32)
    os_ref[0:1, :] += jnp.sum(h4, axis=0, keepdims=True)


def _tc4c_body(h_ref, sum_ref, gb_ref, oss_ref, scr):
    i = pl.program_id(0)
    mu = sum_ref[0:1, :] * (1.0 / N)
    d = h_ref[...] - mu
    ssq = jnp.sum(d * d, axis=0, keepdims=True)

    @pl.when(i == 0)
    def _():
        scr[...] = jnp.zeros((8, 4 * L), jnp.float32)
    scr[0:1, :] += ssq

    @pl.when(i == NG_TC - 1)
    def _():
        var = scr[0:1, :] * (1.0 / N)
        invstd = lax.rsqrt(var + 1e-5)
        scale = gb_ref[0:1, :] * invstd
        shift = gb_ref[1:2, :] - mu * scale
        oss_ref[...] = jnp.concatenate([scale, shift], axis=0)


def _tcp_body(h_ref, ss_ref, b_ref, o_ref):
    i = pl.program_id(0)
    hn = h_ref[...] * ss_ref[0:1, :] + ss_ref[1:2, :]
    hn = jnp.maximum(hn, hn * 0.01)
    bio = lax.broadcasted_iota(jnp.int32, (G, BN_TC), 0)
    ids = b_ref[pl.ds(i, 1), :]
    onehot = (bio == ids).astype(jnp.float32)
    pooled = jnp.dot(onehot, hn, preferred_element_type=jnp.float32)

    @pl.when(i == 0)
    def _():
        o_ref[...] = jnp.zeros((G, 4 * L), jnp.float32)
    o_ref[...] += pooled


def _tc5_body(p_ref, fW1_ref, fb1_ref, fW2_ref, fb2_ref, fW3_ref, fb3_ref,
              o_ref):
    pooled = p_ref[...]
    o = jnp.dot(pooled, fW1_ref[...].T, preferred_element_type=jnp.float32)
    o = _leaky(o + fb1_ref[...])
    o = jnp.dot(o, fW2_ref[...].T, preferred_element_type=jnp.float32)
    o = _leaky(o + fb2_ref[...])
    o = jnp.dot(o, fW3_ref[...].T, preferred_element_type=jnp.float32)
    o_ref[...] = o + fb3_ref[...]


def kernel(x, edge_index, edge_weight, batch, W1, b1, W2, b2, W3, b3, W4, b4,
           gamma, beta, fW1, fb1, fW2, fb2, fW3, fb3):
    f32 = jnp.float32
    src = edge_index[0]
    dst = edge_index[1]
    pad = E2 - E
    fill = (jnp.arange(pad, dtype=jnp.int32) * 37) % N
    src2 = jnp.concatenate([src, fill]).reshape(E2 // CHUNK, CHUNK)
    dst2 = jnp.concatenate([dst, fill]).reshape(E2 // CHUNK, CHUNK)
    ew2 = jnp.concatenate(
        [edge_weight, jnp.zeros((pad,), f32)]).reshape(E2 // CHUNK, CHUNK)

    xp = jnp.pad(x, ((0, 0), (0, L - 6)))
    W1p = jnp.pad(W1, ((0, 0), (0, L - 6)))
    W4p = jnp.pad(W4, ((0, 64 - 50), (0, 0)))
    b4p = jnp.pad(b4, (0, 64 - 50)).reshape(1, 64)
    gb = jnp.stack([jnp.pad(gamma, (0, 64 - 50)),
                    jnp.pad(beta, (0, 64 - 50))])
    fW1p = jnp.pad(fW1, ((0, 0), (0, 64 - 50)))

    agg_k1 = _make_sc_agg(1)
    agg_k2 = _make_sc_agg(2)
    agg_k4 = _make_sc_agg(4)

    grid = NG_TC

    def blk(k):
        return pl.BlockSpec((k, BN_TC, L), lambda i: (0, i, 0))

    def blk2(w):
        return pl.BlockSpec((BN_TC, w), lambda i: (i, 0))

    def full(shape):
        return pl.BlockSpec(shape, lambda i: tuple(0 for _ in shape))

    p1 = agg_k1(xp, src2, dst2, ew2).reshape(2, N, L)
    h1 = pl.pallas_call(
        _tc1_body, grid=(grid,),
        in_specs=[blk(2), full((L, L)), full((1, L))],
        out_specs=blk2(L),
        out_shape=jax.ShapeDtypeStruct((N, L), f32),
    )(p1, W1p, b1.reshape(1, L))

    p2 = agg_k1(h1, src2, dst2, ew2).reshape(2, N, L)
    h2a, h2b = pl.pallas_call(
        _tc2_body, grid=(grid,),
        in_specs=[blk(2), full((32, L)), full((1, 32))],
        out_specs=[blk2(L), blk2(L)],
        out_shape=[jax.ShapeDtypeStruct((N, L), f32)] * 2,
    )(p2, W2, b2.reshape(1, 32))

    p3 = agg_k2(h2a, h2b, src2, dst2, ew2).reshape(4, N, L)
    v4 = pl.pallas_call(
        _tc3_body, grid=(grid,),
        in_specs=[blk(4), full((64, 32)), full((1, 64)), full((64, 64))],
        out_specs=[blk2(L)] * 4,
        out_shape=[jax.ShapeDtypeStruct((N, L), f32)] * 4,
    )(p3, W3, b3.reshape(1, 64), W4p)

    p4 = agg_k4(v4[0], v4[1], v4[2], v4[3], src2, dst2, ew2).reshape(8, N, L)
    h4, sums = pl.pallas_call(
        _tc4_body, grid=(grid,),
        in_specs=[blk(8), full((1, 64))],
        out_specs=[blk2(64), full((8, 64))],
        out_shape=[jax.ShapeDtypeStruct((N, 64), f32),
                   jax.ShapeDtypeStruct((8, 64), f32)],
    )(p4, b4p)

    ss = pl.pallas_call(
        _tc4c_body, grid=(grid,),
        in_specs=[blk2(64), full((8, 64)), full((2, 64))],
        out_specs=full((2, 64)),
        out_shape=jax.ShapeDtypeStruct((2, 64), f32),
        scratch_shapes=[pltpu.VMEM((8, 64), f32)],
    )(h4, sums, gb)

    pooled = pl.pallas_call(
        _tcp_body, grid=(grid,),
        in_specs=[blk2(64), full((2, 64)), full((NG_TC, BN_TC))],
        out_specs=full((G, 64)),
        out_shape=jax.ShapeDtypeStruct((G, 64), f32),
    )(h4, ss, batch.reshape(NG_TC, BN_TC))

    out = pl.pallas_call(
        _tc5_body, grid=(1,),
        in_specs=[full((G, 64)), full((30, 64)), full((1, 30)),
                  full((20, 30)), full((1, 20)), full((2, 20)), full((1, 2))],
        out_specs=full((G, 2)),
        out_shape=jax.ShapeDtypeStruct((G, 2), f32),
    )(pooled, fW1p, fb1.reshape(1, 30), fW2, fb2.reshape(1, 20),
      fW3, fb3.reshape(1, 2))
    return out
